# baseline (speedup 1.0000x reference)
import functools

import jax
import jax.numpy as jnp
from jax import lax
from jax.experimental import pallas as pl
from jax.experimental.pallas import tpu as pltpu
from jax.experimental.pallas import tpu_sc as plsc

_BATCH = 64
_NUM_POS = 196
_HIDDEN = 768
_P_OUT = _NUM_POS + 1

_LANES = 16
_ROWS = 7
_MAIN_WORKERS = 28
_NB = 8
_NGROUPS = _BATCH // _NB
_VECS = _HIDDEN // _LANES


def _body(feats_hbm, qw_hbm, pw_hbm, out_hbm,
          in_buf, out_buf, pw_buf, row0_buf, sem_in, sem_out):
    c = lax.axis_index("c")
    s = lax.axis_index("s")
    wid = s * 2 + c

    @pl.when(wid < _MAIN_WORKERS)
    def _main():
        p0 = wid * _ROWS
        pltpu.sync_copy(pw_hbm.at[pl.ds(p0 + 1, _ROWS), :], pw_buf)

        def group(grp, _):
            b0 = grp * _NB
            pltpu.async_copy(
                feats_hbm.at[pl.ds(b0, _NB), pl.ds(p0, _ROWS), :],
                in_buf, sem_in).wait()

            def vec_step(j, _):
                r = j // _VECS
                co = (j % _VECS) * _LANES
                pwv = pw_buf[r, pl.ds(co, _LANES)]
                for g in range(_NB):
                    out_buf[g, r, pl.ds(co, _LANES)] = (
                        in_buf[g, r, pl.ds(co, _LANES)] + pwv)
                return 0

            lax.fori_loop(0, _ROWS * _VECS, vec_step, 0)
            pltpu.async_copy(
                out_buf,
                out_hbm.at[pl.ds(b0, _NB), pl.ds(p0 + 1, _ROWS), :],
                sem_out).wait()
            return 0

        lax.fori_loop(0, _NGROUPS, group, 0)

    @pl.when(wid >= _MAIN_WORKERS)
    def _row0():
        nb = _BATCH // (32 - _MAIN_WORKERS)
        b0 = (wid - _MAIN_WORKERS) * nb
        pltpu.sync_copy(pw_hbm.at[pl.ds(0, 1), :], pw_buf.at[pl.ds(0, 1), :])
        pltpu.sync_copy(qw_hbm, pw_buf.at[pl.ds(1, 1), :])

        def vec_step(v, _):
            co = v * _LANES
            val = pw_buf[0, pl.ds(co, _LANES)] + pw_buf[1, pl.ds(co, _LANES)]
            for r in range(16):
                row0_buf[r, 0, pl.ds(co, _LANES)] = val
            return 0

        lax.fori_loop(0, _VECS, vec_step, 0)
        pltpu.async_copy(
            row0_buf,
            out_hbm.at[pl.ds(b0, nb), pl.ds(0, 1), :],
            sem_out).wait()


@jax.jit
def kernel(feats, quality_weight, position_weight):
    mesh = plsc.VectorSubcoreMesh(core_axis_name="c", subcore_axis_name="s")
    run = pl.kernel(
        _body,
        out_type=jax.ShapeDtypeStruct((_BATCH, _P_OUT, _HIDDEN), jnp.float32),
        mesh=mesh,
        scratch_types=[
            pltpu.VMEM((_NB, _ROWS, _HIDDEN), jnp.float32),
            pltpu.VMEM((_NB, _ROWS, _HIDDEN), jnp.float32),
            pltpu.VMEM((_ROWS, _HIDDEN), jnp.float32),
            pltpu.VMEM((16, 1, _HIDDEN), jnp.float32),
            pltpu.SemaphoreType.DMA,
            pltpu.SemaphoreType.DMA,
        ],
        compiler_params=pltpu.CompilerParams(use_tc_tiling_on_sc=False),
    )
    return run(feats, quality_weight, position_weight)


# speedup vs baseline: 1.1954x; 1.1954x over previous
import jax
import jax.numpy as jnp
from jax import lax
from jax.experimental import pallas as pl
from jax.experimental.pallas import tpu as pltpu
from jax.experimental.pallas import tpu_sc as plsc

_BATCH = 64
_NUM_POS = 196
_HIDDEN = 768
_P_OUT = _NUM_POS + 1

_LANES = 16
_ROWS = 7
_MAIN_WORKERS = 28
_NB = 4
_NGROUPS = _BATCH // _NB
_NBUF = 3
_VECS = _HIDDEN // _LANES


def _body(feats_hbm, qw_hbm, pw_hbm, out_hbm,
          buf0, buf1, buf2, pw_buf, row0_buf,
          si0, si1, si2, so0, so1, so2):
    bufs = [buf0, buf1, buf2]
    sems_in = [si0, si1, si2]
    sems_out = [so0, so1, so2]
    c = lax.axis_index("c")
    s = lax.axis_index("s")
    wid = s * 2 + c

    @pl.when(wid < _MAIN_WORKERS)
    def _main():
        p0 = wid * _ROWS
        pltpu.sync_copy(pw_hbm.at[pl.ds(p0 + 1, _ROWS), :], pw_buf)

        def in_dma(g):
            return pltpu.async_copy(
                feats_hbm.at[pl.ds(g * _NB, _NB), pl.ds(p0, _ROWS), :],
                bufs[g % _NBUF], sems_in[g % _NBUF])

        def out_dma(g):
            return pltpu.async_copy(
                bufs[g % _NBUF],
                out_hbm.at[pl.ds(g * _NB, _NB), pl.ds(p0 + 1, _ROWS), :],
                sems_out[g % _NBUF])

        def compute(g):
            buf = bufs[g % _NBUF]

            def vec_step(v, _):
                co = v * _LANES
                for r in range(_ROWS):
                    pwv = pw_buf[r, pl.ds(co, _LANES)]
                    for b in range(_NB):
                        buf[b, r, pl.ds(co, _LANES)] = (
                            buf[b, r, pl.ds(co, _LANES)] + pwv)
                return 0

            lax.fori_loop(0, _VECS, vec_step, 0)

        h_in = [None] * _NGROUPS
        h_out = [None] * _NGROUPS
        h_in[0] = in_dma(0)
        for g in range(_NGROUPS):
            if g >= 2:
                h_out[g - 2].wait()
            if g + 1 < _NGROUPS:
                h_in[g + 1] = in_dma(g + 1)
            h_in[g].wait()
            compute(g)
            h_out[g] = out_dma(g)
        h_out[_NGROUPS - 2].wait()
        h_out[_NGROUPS - 1].wait()

    @pl.when(wid >= _MAIN_WORKERS)
    def _row0():
        nb = _BATCH // (32 - _MAIN_WORKERS)
        b0 = (wid - _MAIN_WORKERS) * nb
        pltpu.sync_copy(pw_hbm.at[pl.ds(0, 1), :], pw_buf.at[pl.ds(0, 1), :])
        pltpu.sync_copy(qw_hbm, pw_buf.at[pl.ds(1, 1), :])

        def vec_step(v, _):
            co = v * _LANES
            val = pw_buf[0, pl.ds(co, _LANES)] + pw_buf[1, pl.ds(co, _LANES)]
            for r in range(16):
                row0_buf[r, 0, pl.ds(co, _LANES)] = val
            return 0

        lax.fori_loop(0, _VECS, vec_step, 0)
        pltpu.async_copy(
            row0_buf,
            out_hbm.at[pl.ds(b0, nb), pl.ds(0, 1), :],
            sems_out[0]).wait()


@jax.jit
def kernel(feats, quality_weight, position_weight):
    mesh = plsc.VectorSubcoreMesh(core_axis_name="c", subcore_axis_name="s")
    run = pl.kernel(
        _body,
        out_type=jax.ShapeDtypeStruct((_BATCH, _P_OUT, _HIDDEN), jnp.float32),
        mesh=mesh,
        scratch_types=[
            pltpu.VMEM((_NB, _ROWS, _HIDDEN), jnp.float32),
            pltpu.VMEM((_NB, _ROWS, _HIDDEN), jnp.float32),
            pltpu.VMEM((_NB, _ROWS, _HIDDEN), jnp.float32),
            pltpu.VMEM((_ROWS, _HIDDEN), jnp.float32),
            pltpu.VMEM((16, 1, _HIDDEN), jnp.float32),
            pltpu.SemaphoreType.DMA,
            pltpu.SemaphoreType.DMA,
            pltpu.SemaphoreType.DMA,
            pltpu.SemaphoreType.DMA,
            pltpu.SemaphoreType.DMA,
            pltpu.SemaphoreType.DMA,
        ],
        compiler_params=pltpu.CompilerParams(use_tc_tiling_on_sc=False),
    )
    return run(feats, quality_weight, position_weight)


# speedup vs baseline: 1.4110x; 1.1804x over previous
import jax
import jax.numpy as jnp
from jax import lax
from jax.experimental import pallas as pl
from jax.experimental.pallas import tpu as pltpu
from jax.experimental.pallas import tpu_sc as plsc

_BATCH = 64
_NUM_POS = 196
_HIDDEN = 768
_P_OUT = _NUM_POS + 1

_LANES = 16
_NB = 2
_NGROUPS = _BATCH // _NB
_NBUF = 3
_VECS = _HIDDEN // _LANES


def _body(feats_hbm, qw_hbm, pw_hbm, out_hbm,
          buf0, buf1, buf2, pw_buf, small, tail_out, tail_pw, qw_buf,
          si0, si1, si2, so0, so1, so2):
    bufs = [buf0, buf1, buf2]
    sems_in = [si0, si1, si2]
    sems_out = [so0, so1, so2]
    c = lax.axis_index("c")
    s = lax.axis_index("s")
    wid = s * 2 + c

    @pl.when(jnp.logical_and(wid >= 1, wid < 24))
    def _interior():
        r0 = wid * 8
        pltpu.sync_copy(pw_hbm.at[pl.ds(r0, 8), :], pw_buf)

        def in_dma(g):
            return pltpu.async_copy(
                feats_hbm.at[pl.ds(g * _NB, _NB), pl.ds(r0 - 8, 16), :],
                bufs[g % _NBUF], sems_in[g % _NBUF])

        def out_dma(g):
            return pltpu.async_copy(
                bufs[g % _NBUF].at[:, pl.ds(0, 8), :],
                out_hbm.at[pl.ds(g * _NB, _NB), pl.ds(r0, 8), :],
                sems_out[g % _NBUF])

        def compute(g):
            buf = bufs[g % _NBUF]

            def vec_step(v, _):
                co = v * _LANES
                for r in range(8):
                    pwv = pw_buf[r, pl.ds(co, _LANES)]
                    for b in range(_NB):
                        buf[b, r, pl.ds(co, _LANES)] = (
                            buf[b, r + 7, pl.ds(co, _LANES)] + pwv)
                return 0

            lax.fori_loop(0, _VECS, vec_step, 0)

        h_in = [None] * _NGROUPS
        h_out = [None] * _NGROUPS
        h_in[0] = in_dma(0)
        for g in range(_NGROUPS):
            if g >= 2:
                h_out[g - 2].wait()
            if g + 1 < _NGROUPS:
                h_in[g + 1] = in_dma(g + 1)
            h_in[g].wait()
            compute(g)
            h_out[g] = out_dma(g)
        h_out[_NGROUPS - 2].wait()
        h_out[_NGROUPS - 1].wait()

    @pl.when(wid == 0)
    def _head():
        pltpu.sync_copy(pw_hbm.at[pl.ds(0, 8), :], pw_buf)
        pltpu.sync_copy(qw_hbm, qw_buf)

        def row0_step(v, _):
            co = v * _LANES
            val = qw_buf[0, pl.ds(co, _LANES)] + pw_buf[0, pl.ds(co, _LANES)]
            for b in range(_NB):
                small[b, 0, pl.ds(co, _LANES)] = val
            return 0

        lax.fori_loop(0, _VECS, row0_step, 0)

        def in_dma(g):
            return pltpu.async_copy(
                feats_hbm.at[pl.ds(g * _NB, _NB), pl.ds(0, 8), :],
                bufs[g % _NBUF].at[:, pl.ds(0, 8), :], sems_in[g % _NBUF])

        def out_dma(g):
            return pltpu.async_copy(
                small.at[:, pl.ds(0, 8), :],
                out_hbm.at[pl.ds(g * _NB, _NB), pl.ds(0, 8), :],
                sems_out[0])

        def compute(g):
            buf = bufs[g % _NBUF]

            def vec_step(v, _):
                co = v * _LANES
                for r in range(1, 8):
                    pwv = pw_buf[r, pl.ds(co, _LANES)]
                    for b in range(_NB):
                        small[b, r, pl.ds(co, _LANES)] = (
                            buf[b, r - 1, pl.ds(co, _LANES)] + pwv)
                return 0

            lax.fori_loop(0, _VECS, vec_step, 0)

        h_in = [None] * _NGROUPS
        h_out = [None] * _NGROUPS
        h_in[0] = in_dma(0)
        for g in range(_NGROUPS):
            if g + 1 < _NGROUPS:
                h_in[g + 1] = in_dma(g + 1)
            h_in[g].wait()
            if g >= 1:
                h_out[g - 1].wait()
            compute(g)
            h_out[g] = out_dma(g)
        h_out[_NGROUPS - 1].wait()

    @pl.when(wid == 24)
    def _tail():
        pltpu.sync_copy(pw_hbm.at[pl.ds(192, 5), :], tail_pw)

        def in_dma(g):
            return pltpu.async_copy(
                feats_hbm.at[pl.ds(g * _NB, _NB), pl.ds(184, 12), :],
                small, sems_in[0])

        def out_dma(g):
            return pltpu.async_copy(
                tail_out,
                out_hbm.at[pl.ds(g * _NB, _NB), pl.ds(192, 5), :],
                sems_out[0])

        def compute(g):
            def vec_step(v, _):
                co = v * _LANES
                for r in range(5):
                    pwv = tail_pw[r, pl.ds(co, _LANES)]
                    for b in range(_NB):
                        tail_out[b, r, pl.ds(co, _LANES)] = (
                            small[b, r + 7, pl.ds(co, _LANES)] + pwv)
                return 0

            lax.fori_loop(0, _VECS, vec_step, 0)

        h_in = [None] * _NGROUPS
        h_out = [None] * _NGROUPS
        h_in[0] = in_dma(0)
        for g in range(_NGROUPS):
            h_in[g].wait()
            if g >= 1:
                h_out[g - 1].wait()
            compute(g)
            h_out[g] = out_dma(g)
            if g + 1 < _NGROUPS:
                h_in[g + 1] = in_dma(g + 1)
        h_out[_NGROUPS - 1].wait()


@jax.jit
def kernel(feats, quality_weight, position_weight):
    mesh = plsc.VectorSubcoreMesh(core_axis_name="c", subcore_axis_name="s")
    run = pl.kernel(
        _body,
        out_type=jax.ShapeDtypeStruct((_BATCH, _P_OUT, _HIDDEN), jnp.float32),
        mesh=mesh,
        scratch_types=[
            pltpu.VMEM((_NB, 16, _HIDDEN), jnp.float32),
            pltpu.VMEM((_NB, 16, _HIDDEN), jnp.float32),
            pltpu.VMEM((_NB, 16, _HIDDEN), jnp.float32),
            pltpu.VMEM((8, _HIDDEN), jnp.float32),
            pltpu.VMEM((_NB, 12, _HIDDEN), jnp.float32),
            pltpu.VMEM((_NB, 5, _HIDDEN), jnp.float32),
            pltpu.VMEM((5, _HIDDEN), jnp.float32),
            pltpu.VMEM((1, _HIDDEN), jnp.float32),
            pltpu.SemaphoreType.DMA,
            pltpu.SemaphoreType.DMA,
            pltpu.SemaphoreType.DMA,
            pltpu.SemaphoreType.DMA,
            pltpu.SemaphoreType.DMA,
            pltpu.SemaphoreType.DMA,
        ],
    )
    return run(feats, quality_weight, position_weight)
